# Initial kernel scaffold; baseline (speedup 1.0000x reference)
import jax
import jax.numpy as jnp
from jax.experimental import pallas as pl


def kernel(x, W, b, Wq, Wk, A, Bm):
    raise NotImplementedError("write your pallas kernel here")


# speedup vs baseline: 2.2121x; 2.2121x over previous
import functools
import math

import jax
import jax.numpy as jnp
from jax.experimental import pallas as pl

B, S, IN, OUT, E, R, DK = 2, 4096, 768, 768, 8, 8, 32
SCALING = 16.0 / 8.0
TILE = 512


def _kernel(x_ref, wt_ref, b_ref, wqt_ref, wkt_ref, at_ref, bmf_ref, out_ref):
    xt = x_ref[...]

    result = jnp.dot(xt, wt_ref[...], preferred_element_type=jnp.float32)

    q = jnp.dot(xt, wqt_ref[...], preferred_element_type=jnp.float32)
    k = jnp.dot(xt, wkt_ref[...], preferred_element_type=jnp.float32)
    qk = q * k
    col = jax.lax.broadcasted_iota(jnp.int32, (E * DK, E), 0)
    exp = jax.lax.broadcasted_iota(jnp.int32, (E * DK, E), 1)
    seg = (col // DK == exp).astype(jnp.float32)
    scores = jnp.dot(qk, seg, preferred_element_type=jnp.float32)
    scores = scores * (1.0 / math.sqrt(DK))
    scores = scores - jnp.max(scores, axis=-1, keepdims=True)
    ew = jnp.exp(scores)
    w = ew / jnp.sum(ew, axis=-1, keepdims=True)

    h = jnp.dot(xt, at_ref[...], preferred_element_type=jnp.float32)
    ecol = jax.lax.broadcasted_iota(jnp.int32, (E, E * R), 0)
    ridx = jax.lax.broadcasted_iota(jnp.int32, (E, E * R), 1)
    rep = (ridx // R == ecol).astype(jnp.float32)
    hw = h * jnp.dot(w, rep, preferred_element_type=jnp.float32)
    combined = jnp.dot(hw, bmf_ref[...], preferred_element_type=jnp.float32)

    out_ref[...] = result + b_ref[...] + SCALING * combined


@jax.jit
def kernel(x, W, b, Wq, Wk, A, Bm):
    rows = B * S
    xf = x.reshape(rows, IN)
    wt = W.T
    wqt = Wq.T
    wkt = Wk.T
    at = A.reshape(E * R, IN).T
    bmf = jnp.transpose(Bm, (0, 2, 1)).reshape(E * R, OUT)
    b2 = b.reshape(1, OUT)

    grid = (rows // TILE,)
    out = pl.pallas_call(
        _kernel,
        grid=grid,
        in_specs=[
            pl.BlockSpec((TILE, IN), lambda i: (i, 0)),
            pl.BlockSpec((IN, OUT), lambda i: (0, 0)),
            pl.BlockSpec((1, OUT), lambda i: (0, 0)),
            pl.BlockSpec((IN, E * DK), lambda i: (0, 0)),
            pl.BlockSpec((IN, E * DK), lambda i: (0, 0)),
            pl.BlockSpec((IN, E * R), lambda i: (0, 0)),
            pl.BlockSpec((E * R, OUT), lambda i: (0, 0)),
        ],
        out_specs=pl.BlockSpec((TILE, OUT), lambda i: (i, 0)),
        out_shape=jax.ShapeDtypeStruct((rows, OUT), jnp.float32),
    )(xf, wt, b2, wqt, wkt, at, bmf)
    return out.reshape(B, S, OUT)


# speedup vs baseline: 2.4269x; 1.0971x over previous
import math

import jax
import jax.numpy as jnp
from jax.experimental import pallas as pl

B, S, IN, OUT, E, R, DK = 2, 4096, 768, 768, 8, 8, 32
SCALING = 16.0 / 8.0
TILE = 512
KQ = E * DK


def _kernel(x_ref, wcat_ref, b_ref, segrep_ref, bmf_ref, out_ref):
    xt = x_ref[...]

    big = jnp.dot(xt, wcat_ref[...], preferred_element_type=jnp.float32)
    result = big[:, :OUT]
    q = big[:, OUT:OUT + KQ]
    k = big[:, OUT + KQ:OUT + 2 * KQ]
    h = big[:, OUT + 2 * KQ:]

    s64 = jnp.dot(q * k, segrep_ref[...], preferred_element_type=jnp.float32)
    m = jnp.max(s64, axis=-1, keepdims=True)
    ew = jnp.exp(s64 - m)
    denom = jnp.sum(ew, axis=-1, keepdims=True)
    hw = h * ew * (float(R) / denom)

    combined = jnp.dot(hw, bmf_ref[...], preferred_element_type=jnp.float32)
    out_ref[...] = result + b_ref[...] + combined


@jax.jit
def kernel(x, W, b, Wq, Wk, A, Bm):
    rows = B * S
    xf = x.reshape(rows, IN)
    wcat = jnp.concatenate(
        [W.T, Wq.T, Wk.T, A.reshape(E * R, IN).T], axis=1)
    bmf = jnp.transpose(Bm, (0, 2, 1)).reshape(E * R, OUT) * SCALING
    b2 = b.reshape(1, OUT)
    j = jnp.arange(KQ)[:, None] // DK
    e = jnp.arange(E * R)[None, :] // R
    segrep = (j == e).astype(jnp.float32) * (1.0 / math.sqrt(DK))

    grid = (rows // TILE,)
    out = pl.pallas_call(
        _kernel,
        grid=grid,
        in_specs=[
            pl.BlockSpec((TILE, IN), lambda i: (i, 0)),
            pl.BlockSpec((IN, OUT + 2 * KQ + E * R), lambda i: (0, 0)),
            pl.BlockSpec((1, OUT), lambda i: (0, 0)),
            pl.BlockSpec((KQ, E * R), lambda i: (0, 0)),
            pl.BlockSpec((E * R, OUT), lambda i: (0, 0)),
        ],
        out_specs=pl.BlockSpec((TILE, OUT), lambda i: (i, 0)),
        out_shape=jax.ShapeDtypeStruct((rows, OUT), jnp.float32),
    )(xf, wcat, b2, segrep, bmf)
    return out.reshape(B, S, OUT)


# speedup vs baseline: 2.5250x; 1.0404x over previous
import math

import jax
import jax.numpy as jnp
from jax.experimental import pallas as pl

B, S, IN, OUT, E, R, DK = 2, 4096, 768, 768, 8, 8, 32
SCALING = 16.0 / 8.0
TILE = 512
KQ = E * DK


def _kernel(x_ref, wcat_ref, b_ref, segrep_ref, bmf_ref, out_ref):
    xt = x_ref[...]
    xb = xt.astype(jnp.bfloat16)

    big = jnp.dot(xb, wcat_ref[...], preferred_element_type=jnp.float32)
    result = big[:, :OUT]
    q = big[:, OUT:OUT + KQ]
    k = big[:, OUT + KQ:OUT + 2 * KQ]
    h = big[:, OUT + 2 * KQ:]

    qk = (q * k).astype(jnp.bfloat16)
    s64 = jnp.dot(qk, segrep_ref[...], preferred_element_type=jnp.float32)
    m = jnp.max(s64, axis=-1, keepdims=True)
    ew = jnp.exp(s64 - m)
    denom = jnp.sum(ew, axis=-1, keepdims=True)
    hw = (h * ew * (float(R) / denom)).astype(jnp.bfloat16)

    combined = jnp.dot(hw, bmf_ref[...], preferred_element_type=jnp.float32)
    out_ref[...] = result + b_ref[...] + combined


@jax.jit
def kernel(x, W, b, Wq, Wk, A, Bm):
    rows = B * S
    xf = x.reshape(rows, IN)
    wcat = jnp.concatenate(
        [W.T, Wq.T, Wk.T, A.reshape(E * R, IN).T],
        axis=1).astype(jnp.bfloat16)
    bmf = (jnp.transpose(Bm, (0, 2, 1)).reshape(E * R, OUT)
           * SCALING).astype(jnp.bfloat16)
    b2 = b.reshape(1, OUT)
    j = jnp.arange(KQ)[:, None] // DK
    e = jnp.arange(E * R)[None, :] // R
    segrep = ((j == e).astype(jnp.float32)
              * (1.0 / math.sqrt(DK))).astype(jnp.bfloat16)

    grid = (rows // TILE,)
    out = pl.pallas_call(
        _kernel,
        grid=grid,
        in_specs=[
            pl.BlockSpec((TILE, IN), lambda i: (i, 0)),
            pl.BlockSpec((IN, OUT + 2 * KQ + E * R), lambda i: (0, 0)),
            pl.BlockSpec((1, OUT), lambda i: (0, 0)),
            pl.BlockSpec((KQ, E * R), lambda i: (0, 0)),
            pl.BlockSpec((E * R, OUT), lambda i: (0, 0)),
        ],
        out_specs=pl.BlockSpec((TILE, OUT), lambda i: (i, 0)),
        out_shape=jax.ShapeDtypeStruct((rows, OUT), jnp.float32),
    )(xf, wcat, b2, segrep, bmf)
    return out.reshape(B, S, OUT)


# speedup vs baseline: 2.7582x; 1.0923x over previous
import math

import jax
import jax.numpy as jnp
from jax.experimental import pallas as pl

B, S, IN, OUT, E, R, DK = 2, 4096, 768, 768, 8, 8, 32
SCALING = 16.0 / 8.0
TILE = 1024
KQ = E * DK


def _kernel(x_ref, wcat_ref, b_ref, segrep_ref, bmf_ref, out_ref):
    xt = x_ref[...]
    xb = xt.astype(jnp.bfloat16)

    big = jnp.dot(xb, wcat_ref[...], preferred_element_type=jnp.float32)
    result = big[:, :OUT]
    q = big[:, OUT:OUT + KQ]
    k = big[:, OUT + KQ:OUT + 2 * KQ]
    h = big[:, OUT + 2 * KQ:]

    qk = (q * k).astype(jnp.bfloat16)
    s64 = jnp.dot(qk, segrep_ref[...], preferred_element_type=jnp.float32)
    m = jnp.max(s64, axis=-1, keepdims=True)
    ew = jnp.exp(s64 - m)
    denom = jnp.sum(ew, axis=-1, keepdims=True)
    hw = (h * ew * (float(R) / denom)).astype(jnp.bfloat16)

    combined = jnp.dot(hw, bmf_ref[...], preferred_element_type=jnp.float32)
    out_ref[...] = result + b_ref[...] + combined


@jax.jit
def kernel(x, W, b, Wq, Wk, A, Bm):
    rows = B * S
    xf = x.reshape(rows, IN)
    wcat = jnp.concatenate(
        [W.T, Wq.T, Wk.T, A.reshape(E * R, IN).T],
        axis=1).astype(jnp.bfloat16)
    bmf = (jnp.transpose(Bm, (0, 2, 1)).reshape(E * R, OUT)
           * SCALING).astype(jnp.bfloat16)
    b2 = b.reshape(1, OUT)
    j = jnp.arange(KQ)[:, None] // DK
    e = jnp.arange(E * R)[None, :] // R
    segrep = ((j == e).astype(jnp.float32)
              * (1.0 / math.sqrt(DK))).astype(jnp.bfloat16)

    grid = (rows // TILE,)
    out = pl.pallas_call(
        _kernel,
        grid=grid,
        in_specs=[
            pl.BlockSpec((TILE, IN), lambda i: (i, 0)),
            pl.BlockSpec((IN, OUT + 2 * KQ + E * R), lambda i: (0, 0)),
            pl.BlockSpec((1, OUT), lambda i: (0, 0)),
            pl.BlockSpec((KQ, E * R), lambda i: (0, 0)),
            pl.BlockSpec((E * R, OUT), lambda i: (0, 0)),
        ],
        out_specs=pl.BlockSpec((TILE, OUT), lambda i: (i, 0)),
        out_shape=jax.ShapeDtypeStruct((rows, OUT), jnp.float32),
    )(xf, wcat, b2, segrep, bmf)
    return out.reshape(B, S, OUT)


# speedup vs baseline: 2.7765x; 1.0066x over previous
import math

import jax
import jax.numpy as jnp
from jax.experimental import pallas as pl

B, S, IN, OUT, E, R, DK = 2, 4096, 768, 768, 8, 8, 32
SCALING = 16.0 / 8.0
TILE = 2048
KQ = E * DK


def _kernel(x_ref, wcat_ref, b_ref, segrep_ref, bmf_ref, out_ref):
    xt = x_ref[...]
    xb = xt.astype(jnp.bfloat16)

    big = jnp.dot(xb, wcat_ref[...], preferred_element_type=jnp.float32)
    result = big[:, :OUT]
    q = big[:, OUT:OUT + KQ]
    k = big[:, OUT + KQ:OUT + 2 * KQ]
    h = big[:, OUT + 2 * KQ:]

    qk = (q * k).astype(jnp.bfloat16)
    s64 = jnp.dot(qk, segrep_ref[...], preferred_element_type=jnp.float32)
    m = jnp.max(s64, axis=-1, keepdims=True)
    ew = jnp.exp(s64 - m)
    denom = jnp.sum(ew, axis=-1, keepdims=True)
    hw = (h * ew * (float(R) / denom)).astype(jnp.bfloat16)

    combined = jnp.dot(hw, bmf_ref[...], preferred_element_type=jnp.float32)
    out_ref[...] = result + b_ref[...] + combined


@jax.jit
def kernel(x, W, b, Wq, Wk, A, Bm):
    rows = B * S
    xf = x.reshape(rows, IN)
    wcat = jnp.concatenate(
        [W.T, Wq.T, Wk.T, A.reshape(E * R, IN).T],
        axis=1).astype(jnp.bfloat16)
    bmf = (jnp.transpose(Bm, (0, 2, 1)).reshape(E * R, OUT)
           * SCALING).astype(jnp.bfloat16)
    b2 = b.reshape(1, OUT)
    j = jnp.arange(KQ)[:, None] // DK
    e = jnp.arange(E * R)[None, :] // R
    segrep = ((j == e).astype(jnp.float32)
              * (1.0 / math.sqrt(DK))).astype(jnp.bfloat16)

    grid = (rows // TILE,)
    out = pl.pallas_call(
        _kernel,
        grid=grid,
        in_specs=[
            pl.BlockSpec((TILE, IN), lambda i: (i, 0)),
            pl.BlockSpec((IN, OUT + 2 * KQ + E * R), lambda i: (0, 0)),
            pl.BlockSpec((1, OUT), lambda i: (0, 0)),
            pl.BlockSpec((KQ, E * R), lambda i: (0, 0)),
            pl.BlockSpec((E * R, OUT), lambda i: (0, 0)),
        ],
        out_specs=pl.BlockSpec((TILE, OUT), lambda i: (i, 0)),
        out_shape=jax.ShapeDtypeStruct((rows, OUT), jnp.float32),
    )(xf, wcat, b2, segrep, bmf)
    return out.reshape(B, S, OUT)
